# Initial kernel scaffold; baseline (speedup 1.0000x reference)
import jax
import jax.numpy as jnp
from jax.experimental import pallas as pl


def kernel(x, edge_index, W1, b1, W2, b2, W3, b3, L1w, L1b, L2w, L2b):
    raise NotImplementedError("write your pallas kernel here")


# speedup vs baseline: 4.5479x; 4.5479x over previous
import functools

import jax
import jax.numpy as jnp
from jax import lax
from jax.experimental import pallas as pl
from jax.experimental.pallas import tpu as pltpu
from jax.experimental.pallas import tpu_sc as plsc

NC = 2
NS = 16
NW = NC * NS
CH = 128


def _mesh():
    return plsc.VectorSubcoreMesh(core_axis_name="c", subcore_axis_name="s")


def _sc_degree(dstp, n, acc_n, cpt):
    del n
    @functools.partial(
        pl.kernel,
        out_type=jax.ShapeDtypeStruct((NC, acc_n), jnp.float32),
        mesh=_mesh(),
        scratch_types=[
            pltpu.VMEM((cpt, CH), jnp.int32),
            pltpu.VMEM((CH,), jnp.float32),
            pltpu.VMEM((acc_n // NS,), jnp.float32),
            pltpu.VMEM_SHARED((acc_n,), jnp.float32),
        ],
    )
    def k(dst_hbm, out_hbm, idx_v, ones_v, z_v, deg_sh):
        c = lax.axis_index("c")
        s = lax.axis_index("s")
        w = s * NC + c
        zslice = acc_n // NS

        def fill_ones(i, _):
            ones_v[pl.ds(i * 16, 16)] = jnp.ones((16,), jnp.float32)
            return 0

        lax.fori_loop(0, CH // 16, fill_ones, 0)

        def fill_zeros(i, _):
            z_v[pl.ds(i * 16, 16)] = jnp.zeros((16,), jnp.float32)
            return 0

        lax.fori_loop(0, zslice // 16, fill_zeros, 0)

        pltpu.sync_copy(z_v, deg_sh.at[pl.ds(s * zslice, zslice)])
        pltpu.sync_copy(dst_hbm.at[w], idx_v)
        plsc.subcore_barrier()

        def body(j, _):
            pltpu.sync_copy(ones_v, deg_sh.at[idx_v.at[j]], add=True)
            return 0

        lax.fori_loop(0, cpt, body, 0)
        plsc.subcore_barrier()

        @pl.when(s == 0)
        def _():
            pltpu.sync_copy(deg_sh.at[pl.ds(0, acc_n)], out_hbm.at[c])

    return k(dstp)


def _sc_aggregate(ts, srcp, dstA, dstB, acc_n, half_n, half_acc, cpt):
    d = ts.shape[1]

    @functools.partial(
        pl.kernel,
        out_type=jax.ShapeDtypeStruct((NC, acc_n, d), jnp.float32),
        mesh=_mesh(),
        scratch_types=[
            pltpu.VMEM((cpt, CH), jnp.int32),
            pltpu.VMEM((cpt, CH), jnp.int32),
            pltpu.VMEM((2, CH, d), jnp.float32),
            pltpu.VMEM((CH, d), jnp.float32),
            pltpu.VMEM_SHARED((half_acc, d), jnp.float32),
            pltpu.SemaphoreType.DMA((2,)),
        ],
    )
    def k(ts_hbm, src_hbm, dstA_hbm, dstB_hbm, out_hbm,
          si_v, di_v, rows_v, z_v, acc_sh, gsem):
        c = lax.axis_index("c")
        s = lax.axis_index("s")
        w = s * NC + c
        zrows_per_tile = half_acc // NS
        orows_per_tile = half_n // NS

        def zrow(r, _):
            def zcol(kk, _):
                z_v[r, pl.ds(kk * 16, 16)] = jnp.zeros((16,), jnp.float32)
                return 0

            lax.fori_loop(0, d // 16, zcol, 0)
            return 0

        lax.fori_loop(0, CH, zrow, 0)
        pltpu.sync_copy(src_hbm.at[w], si_v)

        for half, dst_hbm in ((0, dstA_hbm), (1, dstB_hbm)):
            def zblk(i, _):
                pltpu.sync_copy(z_v, acc_sh.at[pl.ds(s * zrows_per_tile + i * CH, CH)])
                return 0

            lax.fori_loop(0, zrows_per_tile // CH, zblk, 0)
            pltpu.sync_copy(dst_hbm.at[w], di_v)
            plsc.subcore_barrier()

            pltpu.make_async_copy(
                ts_hbm.at[si_v.at[0]], rows_v.at[0], gsem.at[0]
            ).start()

            def body(i, _):
                j0 = i * 2
                pltpu.make_async_copy(
                    ts_hbm.at[si_v.at[j0 + 1]], rows_v.at[1], gsem.at[1]
                ).start()
                pltpu.make_async_copy(
                    ts_hbm.at[si_v.at[j0]], rows_v.at[0], gsem.at[0]
                ).wait()
                pltpu.sync_copy(rows_v.at[0], acc_sh.at[di_v.at[j0]], add=True)

                @pl.when(j0 + 2 < cpt)
                def _():
                    pltpu.make_async_copy(
                        ts_hbm.at[si_v.at[j0 + 2]], rows_v.at[0], gsem.at[0]
                    ).start()

                pltpu.make_async_copy(
                    ts_hbm.at[si_v.at[j0 + 1]], rows_v.at[1], gsem.at[1]
                ).wait()
                pltpu.sync_copy(rows_v.at[1], acc_sh.at[di_v.at[j0 + 1]], add=True)
                return 0

            lax.fori_loop(0, cpt // 2, body, 0)
            plsc.subcore_barrier()
            pltpu.sync_copy(
                acc_sh.at[pl.ds(s * orows_per_tile, orows_per_tile)],
                out_hbm.at[c, pl.ds(half * half_n + s * orows_per_tile,
                                    orows_per_tile)],
            )
            plsc.subcore_barrier()

    return k(ts, srcp, dstA, dstB)


def _tc_prep(deg2, dstp, n, half_n, half_acc):
    acc_n = deg2.shape[1]
    nw, cpt, ch = dstp.shape
    flat = (nw * cpt, ch)
    dstf = dstp.reshape(flat)

    def body(deg_ref, dst_ref, dinv_ref, dstA_ref, dstB_ref):
        dinv_ref[...] = lax.rsqrt(deg_ref[0:1, :] + deg_ref[1:2, :] + 1.0)
        dst = dst_ref[...]
        dump = half_n + lax.rem(
            jax.lax.broadcasted_iota(jnp.int32, flat, 1), half_acc - half_n
        )
        dstA_ref[...] = jnp.where(dst < half_n, dst, dump)
        inB = jnp.logical_and(dst >= half_n, dst < n)
        dstB_ref[...] = jnp.where(inB, dst - half_n, dump)

    dinv, dstA, dstB = pl.pallas_call(
        body,
        out_shape=[
            jax.ShapeDtypeStruct((1, acc_n), jnp.float32),
            jax.ShapeDtypeStruct(flat, jnp.int32),
            jax.ShapeDtypeStruct(flat, jnp.int32),
        ],
    )(deg2, dstf)
    return dinv, dstA.reshape(nw, cpt, ch), dstB.reshape(nw, cpt, ch)


def _tc_first(x, W1, dinv, blk):
    n, d = x.shape

    def body(x_ref, w_ref, dv_ref, out_ref):
        out_ref[...] = (
            jnp.dot(x_ref[...], w_ref[...], preferred_element_type=jnp.float32)
            * dv_ref[...]
        )

    return pl.pallas_call(
        body,
        grid=(n // blk,),
        in_specs=[
            pl.BlockSpec((blk, d), lambda i: (i, 0)),
            pl.BlockSpec((d, d), lambda i: (0, 0)),
            pl.BlockSpec((blk, 1), lambda i: (i, 0)),
        ],
        out_specs=pl.BlockSpec((blk, d), lambda i: (i, 0)),
        out_shape=jax.ShapeDtypeStruct((n, d), jnp.float32),
    )(x, W1, dinv)


def _tc_layer(p, ts, dinv, b, W, blk):
    n, d = ts.shape

    def body(p_ref, ts_ref, dv_ref, b_ref, w_ref, out_ref):
        h = (p_ref[0] + p_ref[1] + ts_ref[...]) * dv_ref[...] + b_ref[...]
        h = jnp.maximum(h, 0.0)
        out_ref[...] = (
            jnp.dot(h, w_ref[...], preferred_element_type=jnp.float32) * dv_ref[...]
        )

    return pl.pallas_call(
        body,
        grid=(n // blk,),
        in_specs=[
            pl.BlockSpec((2, blk, d), lambda i: (0, i, 0)),
            pl.BlockSpec((blk, d), lambda i: (i, 0)),
            pl.BlockSpec((blk, 1), lambda i: (i, 0)),
            pl.BlockSpec((1, d), lambda i: (0, 0)),
            pl.BlockSpec((d, d), lambda i: (0, 0)),
        ],
        out_specs=pl.BlockSpec((blk, d), lambda i: (i, 0)),
        out_shape=jax.ShapeDtypeStruct((n, d), jnp.float32),
    )(p, ts, dinv, b, W)


def _tc_head(p, ts, dinv, b3, L1w, L1b, L2w, L2b, blk):
    n, d = ts.shape
    g_steps = n // blk

    def body(p_ref, ts_ref, dv_ref, b_ref, l1w_ref, l1b_ref, l2w_ref, l2b_ref,
             out_ref, acc_ref):
        i = pl.program_id(0)
        h = (p_ref[0] + p_ref[1] + ts_ref[...]) * dv_ref[...] + b_ref[...]
        h = jnp.maximum(h, 0.0)
        bsum = jnp.sum(h, axis=0, keepdims=True)

        @pl.when(i == 0)
        def _():
            acc_ref[...] = bsum

        @pl.when(i > 0)
        def _():
            acc_ref[...] = acc_ref[...] + bsum

        @pl.when(i == g_steps - 1)
        def _():
            g = acc_ref[...] * (1.0 / n)
            z = jnp.dot(g, l1w_ref[...], preferred_element_type=jnp.float32)
            z = jnp.maximum(z + l1b_ref[...], 0.0)
            o = jnp.dot(z, l2w_ref[...], preferred_element_type=jnp.float32)
            out_ref[...] = jax.nn.sigmoid(o + l2b_ref[...])

    return pl.pallas_call(
        body,
        grid=(g_steps,),
        in_specs=[
            pl.BlockSpec((2, blk, d), lambda i: (0, i, 0)),
            pl.BlockSpec((blk, d), lambda i: (i, 0)),
            pl.BlockSpec((blk, 1), lambda i: (i, 0)),
            pl.BlockSpec((1, d), lambda i: (0, 0)),
            pl.BlockSpec((d, d), lambda i: (0, 0)),
            pl.BlockSpec((1, d), lambda i: (0, 0)),
            pl.BlockSpec((d, 1), lambda i: (0, 0)),
            pl.BlockSpec((1, 1), lambda i: (0, 0)),
        ],
        out_specs=pl.BlockSpec((1, 1), lambda i: (0, 0)),
        out_shape=jax.ShapeDtypeStruct((1, 1), jnp.float32),
        scratch_shapes=[pltpu.VMEM((1, d), jnp.float32)],
    )(p, ts, dinv, b3, L1w, L1b, L2w, L2b)


def kernel(x, edge_index, W1, b1, W2, b2, W3, b3, L1w, L1b, L2w, L2b):
    n, d = x.shape
    e = edge_index.shape[1]
    blk = 2000

    ept = ((e + NW * CH - 1) // (NW * CH)) * CH
    if (ept // CH) % 2:
        ept += CH
    cpt = ept // CH
    epad = NW * ept
    acc_n = ((n + NS * CH) // (NS * CH)) * NS * CH

    src = edge_index[0]
    dst = edge_index[1]
    pad = epad - e
    srcp = jnp.concatenate([src, jnp.zeros((pad,), jnp.int32)]).reshape(NW, cpt, CH)
    dstp = jnp.concatenate([dst, jnp.full((pad,), n, jnp.int32)]).reshape(NW, cpt, CH)

    half_n = acc_n // 2
    half_acc = ((half_n + 1 + NS * CH - 1) // (NS * CH)) * (NS * CH)

    deg2 = _sc_degree(dstp, n, acc_n, cpt)
    dinv, dstA, dstB = _tc_prep(deg2, dstp, n, half_n, half_acc)
    dinv = dinv.reshape(acc_n, 1)

    ts = _tc_first(x, W1, dinv, blk)
    p = _sc_aggregate(ts, srcp, dstA, dstB, acc_n, half_n, half_acc, cpt)
    ts = _tc_layer(p, ts, dinv, b1.reshape(1, d), W2, blk)
    p = _sc_aggregate(ts, srcp, dstA, dstB, acc_n, half_n, half_acc, cpt)
    ts = _tc_layer(p, ts, dinv, b2.reshape(1, d), W3, blk)
    p = _sc_aggregate(ts, srcp, dstA, dstB, acc_n, half_n, half_acc, cpt)
    out = _tc_head(p, ts, dinv, b3.reshape(1, d), L1w, L1b.reshape(1, d),
                   L2w, L2b.reshape(1, 1), blk)
    return out.reshape(1)
